# Initial kernel scaffold; baseline (speedup 1.0000x reference)
import jax
import jax.numpy as jnp
from jax.experimental import pallas as pl


def kernel(y_support, n_obs_query, y_embedding, y_mask):
    raise NotImplementedError("write your pallas kernel here")


# speedup vs baseline: 3.4196x; 3.4196x over previous
import functools

import jax
import jax.numpy as jnp
from jax import lax
from jax.experimental import pallas as pl
from jax.experimental.pallas import tpu as pltpu
from jax.experimental.pallas import tpu_sc as plsc

_DIM = 128
_B = 1024
_S = 200
_Q = 512

_NC = 2
_NS = 16
_NW = _NC * _NS
_CHUNK = 128

_TOTAL = _B * _S
_ROWS_PER_W = _TOTAL // _NW
_CHUNKS_PER_W = _ROWS_PER_W // _CHUNK


def _make_sc_gather():
  mesh = plsc.VectorSubcoreMesh(core_axis_name="c", subcore_axis_name="s")

  @functools.partial(
      pl.kernel,
      mesh=mesh,
      out_type=jax.ShapeDtypeStruct((_TOTAL, _DIM), jnp.float32),
      scratch_types=[
          pltpu.VMEM((_ROWS_PER_W,), jnp.int32),
          pltpu.VMEM((_CHUNK, _DIM), jnp.float32),
          pltpu.SemaphoreType.DMA,
      ],
  )
  def sc_gather(table_hbm, idx_hbm, out_hbm, idx_v, rows_v, sem):
    wid = lax.axis_index("s") * _NC + lax.axis_index("c")
    out_base = wid * _ROWS_PER_W
    pltpu.sync_copy(idx_hbm.at[pl.ds(out_base, _ROWS_PER_W)], idx_v)

    def shift_body(j, carry):
      v = idx_v[pl.ds(j * 16, 16)]
      idx_v[pl.ds(j * 16, 16)] = v + 1
      return carry

    lax.fori_loop(0, _ROWS_PER_W // 16, shift_body, 0)

    def chunk_body(c, carry):
      pltpu.async_copy(
          table_hbm.at[idx_v.at[pl.ds(c * _CHUNK, _CHUNK)]], rows_v, sem
      ).wait()
      pltpu.sync_copy(rows_v, out_hbm.at[pl.ds(out_base + c * _CHUNK, _CHUNK)])
      return carry

    lax.fori_loop(0, _CHUNKS_PER_W, chunk_body, 0)

  return sc_gather


_sc_gather = _make_sc_gather()

_BCAST_ROWS = 8192
_BCAST_GRID = (_B * _Q) // _BCAST_ROWS


def _bcast_body(mask_ref, out_ref):
  out_ref[...] = jnp.broadcast_to(mask_ref[...], out_ref.shape)


_bcast = pl.pallas_call(
    _bcast_body,
    grid=(_BCAST_GRID,),
    in_specs=[pl.BlockSpec((1, _DIM), lambda i: (0, 0))],
    out_specs=pl.BlockSpec((_BCAST_ROWS, _DIM), lambda i: (i, 0)),
    out_shape=jax.ShapeDtypeStruct((_B * _Q, _DIM), jnp.float32),
)


def kernel(y_support, n_obs_query, y_embedding, y_mask):
  del n_obs_query
  idx_flat = y_support.reshape(_TOTAL)
  emb = _sc_gather(y_embedding, idx_flat)
  y_query = _bcast(y_mask)
  return (emb.reshape(_B, _S, _DIM), y_query.reshape(_B, _Q, _DIM))


# speedup vs baseline: 3.4638x; 1.0129x over previous
import functools

import jax
import jax.numpy as jnp
from jax import lax
from jax.experimental import pallas as pl
from jax.experimental.pallas import tpu as pltpu
from jax.experimental.pallas import tpu_sc as plsc

_DIM = 128
_B = 1024
_S = 200
_Q = 512

_NC = 2
_NS = 16
_NW = _NC * _NS
_CHUNK = 128

_TOTAL = _B * _S
_ROWS_PER_W = _TOTAL // _NW
_CHUNKS_PER_W = _ROWS_PER_W // _CHUNK


def _make_sc_gather():
  mesh = plsc.VectorSubcoreMesh(core_axis_name="c", subcore_axis_name="s")

  @functools.partial(
      pl.kernel,
      mesh=mesh,
      out_type=jax.ShapeDtypeStruct((_TOTAL, _DIM), jnp.float32),
      scratch_types=[
          pltpu.VMEM((_ROWS_PER_W,), jnp.int32),
          pltpu.VMEM((_CHUNK, _DIM), jnp.float32),
          pltpu.VMEM((_CHUNK, _DIM), jnp.float32),
          pltpu.SemaphoreType.DMA,
          pltpu.SemaphoreType.DMA,
      ],
  )
  def sc_gather(table_hbm, idx_hbm, out_hbm, idx_v, rows_a, rows_b, sem_a,
                sem_b):
    wid = lax.axis_index("s") * _NC + lax.axis_index("c")
    out_base = wid * _ROWS_PER_W
    pltpu.sync_copy(idx_hbm.at[pl.ds(out_base, _ROWS_PER_W)], idx_v)

    def shift_body(j, carry):
      v = idx_v[pl.ds(j * 16, 16)]
      idx_v[pl.ds(j * 16, 16)] = v + 1
      return carry

    lax.fori_loop(0, _ROWS_PER_W // 16, shift_body, 0)

    def start_gather(c, buf, sem):
      pltpu.async_copy(
          table_hbm.at[idx_v.at[pl.ds(c * _CHUNK, _CHUNK)]], buf, sem
      )

    def drain(buf, sem):
      pltpu.make_async_copy(table_hbm.at[pl.ds(0, _CHUNK)], buf, sem).wait()

    def write_out(c, buf):
      pltpu.sync_copy(buf, out_hbm.at[pl.ds(out_base + c * _CHUNK, _CHUNK)])

    start_gather(0, rows_a, sem_a)

    def pair_body(i, carry):
      c0 = 2 * i
      start_gather(c0 + 1, rows_b, sem_b)
      drain(rows_a, sem_a)
      write_out(c0, rows_a)

      @pl.when(i < _CHUNKS_PER_W // 2 - 1)
      def _():
        start_gather(c0 + 2, rows_a, sem_a)

      drain(rows_b, sem_b)
      write_out(c0 + 1, rows_b)
      return carry

    lax.fori_loop(0, _CHUNKS_PER_W // 2, pair_body, 0)

  return sc_gather


_sc_gather = _make_sc_gather()

_BCAST_ROWS = 8192
_BCAST_GRID = (_B * _Q) // _BCAST_ROWS


def _bcast_body(mask_ref, out_ref):
  out_ref[...] = jnp.broadcast_to(mask_ref[...], out_ref.shape)


_bcast = pl.pallas_call(
    _bcast_body,
    grid=(_BCAST_GRID,),
    in_specs=[pl.BlockSpec((1, _DIM), lambda i: (0, 0))],
    out_specs=pl.BlockSpec((_BCAST_ROWS, _DIM), lambda i: (i, 0)),
    out_shape=jax.ShapeDtypeStruct((_B * _Q, _DIM), jnp.float32),
)


def kernel(y_support, n_obs_query, y_embedding, y_mask):
  del n_obs_query
  idx_flat = y_support.reshape(_TOTAL)
  emb = _sc_gather(y_embedding, idx_flat)
  y_query = _bcast(y_mask)
  return (emb.reshape(_B, _S, _DIM), y_query.reshape(_B, _Q, _DIM))


# speedup vs baseline: 6.2600x; 1.8072x over previous
import functools

import jax
import jax.numpy as jnp
from jax import lax
from jax.experimental import pallas as pl
from jax.experimental.pallas import tpu as pltpu
from jax.experimental.pallas import tpu_sc as plsc

_DIM = 128
_B = 1024
_S = 200
_Q = 512

_NC = 2
_NS = 16
_NW = _NC * _NS
_CHUNK = 128

_TOTAL = _B * _S
_ROWS_PER_W = _TOTAL // _NW
_CHUNKS_PER_W = _ROWS_PER_W // _CHUNK


def _make_sc_gather():
  mesh = plsc.VectorSubcoreMesh(core_axis_name="c", subcore_axis_name="s")

  @functools.partial(
      pl.kernel,
      mesh=mesh,
      out_type=jax.ShapeDtypeStruct((_TOTAL, _DIM), jnp.float32),
      scratch_types=[
          pltpu.VMEM((_ROWS_PER_W,), jnp.int32),
          pltpu.VMEM((_CHUNK, _DIM), jnp.float32),
          pltpu.VMEM((_CHUNK, _DIM), jnp.float32),
          pltpu.VMEM_SHARED((1001, _DIM), jnp.float32),
          pltpu.SemaphoreType.DMA,
          pltpu.SemaphoreType.DMA,
      ],
  )
  def sc_gather(table_hbm, idx_hbm, out_hbm, idx_v, rows_a, rows_b,
                table_sp, sem_a, sem_b):
    sid = lax.axis_index("s")
    wid = sid * _NC + lax.axis_index("c")
    out_base = wid * _ROWS_PER_W

    @pl.when(sid == 0)
    def _():
      pltpu.sync_copy(table_hbm, table_sp)

    pltpu.sync_copy(idx_hbm.at[pl.ds(out_base, _ROWS_PER_W)], idx_v)
    plsc.subcore_barrier()

    def shift_body(j, carry):
      v = idx_v[pl.ds(j * 16, 16)]
      idx_v[pl.ds(j * 16, 16)] = v + 1
      return carry

    lax.fori_loop(0, _ROWS_PER_W // 16, shift_body, 0)

    def start_gather(c, buf, sem):
      pltpu.async_copy(
          table_sp.at[idx_v.at[pl.ds(c * _CHUNK, _CHUNK)]], buf, sem
      )

    def drain(buf, sem):
      pltpu.make_async_copy(table_hbm.at[pl.ds(0, _CHUNK)], buf, sem).wait()

    def write_out(c, buf):
      pltpu.sync_copy(buf, out_hbm.at[pl.ds(out_base + c * _CHUNK, _CHUNK)])

    start_gather(0, rows_a, sem_a)

    def pair_body(i, carry):
      c0 = 2 * i
      start_gather(c0 + 1, rows_b, sem_b)
      drain(rows_a, sem_a)
      write_out(c0, rows_a)

      @pl.when(i < _CHUNKS_PER_W // 2 - 1)
      def _():
        start_gather(c0 + 2, rows_a, sem_a)

      drain(rows_b, sem_b)
      write_out(c0 + 1, rows_b)
      return carry

    lax.fori_loop(0, _CHUNKS_PER_W // 2, pair_body, 0)

  return sc_gather


_sc_gather = _make_sc_gather()

_BCAST_ROWS = 8192
_BCAST_GRID = (_B * _Q) // _BCAST_ROWS


def _bcast_body(mask_ref, out_ref):
  out_ref[...] = jnp.broadcast_to(mask_ref[...], out_ref.shape)


_bcast = pl.pallas_call(
    _bcast_body,
    grid=(_BCAST_GRID,),
    in_specs=[pl.BlockSpec((1, _DIM), lambda i: (0, 0))],
    out_specs=pl.BlockSpec((_BCAST_ROWS, _DIM), lambda i: (i, 0)),
    out_shape=jax.ShapeDtypeStruct((_B * _Q, _DIM), jnp.float32),
)


def kernel(y_support, n_obs_query, y_embedding, y_mask):
  del n_obs_query
  idx_flat = y_support.reshape(_TOTAL)
  emb = _sc_gather(y_embedding, idx_flat)
  y_query = _bcast(y_mask)
  return (emb.reshape(_B, _S, _DIM), y_query.reshape(_B, _Q, _DIM))
